# baseline (device time: 98372 ns/iter reference)
import jax
import jax.numpy as jnp
from jax import lax
from jax.experimental import pallas as pl
from jax.experimental.pallas import tpu as pltpu

N_DEV = 8
CW, CCW = 0, 1
SUB = 4
RING = (0, 1, 2, 3, 7, 6, 5, 4)


def kernel(x, dy):
    m, d = x.shape
    _, f = dy.shape
    d_per = d // N_DEV
    f_half = f // 2
    f_sub = f_half // SUB

    def body(x_ref, dy_ref, out_ref, comm_ref, send_sems, recv_sems):
        my = lax.axis_index("i")

        idx = lax.broadcasted_iota(jnp.int32, (1, N_DEV), 1)
        ring_arr = jnp.where(idx < 4, idx, 11 - idx)
        pos = jnp.sum(jnp.where(ring_arr == my, idx, 0))

        def ring_at(k):
            return jnp.sum(jnp.where(idx == (k % N_DEV), ring_arr, 0))

        right = ring_at(pos + 1)
        left = ring_at(pos - 1)

        barrier_sem = pltpu.get_barrier_semaphore()
        for nbr in (left, right):
            pl.semaphore_signal(
                barrier_sem, inc=1,
                device_id=(nbr,), device_id_type=pl.DeviceIdType.MESH,
            )
        pl.semaphore_wait(barrier_sem, 2)

        def contrib(c, direction):
            xs = x_ref[:, pl.ds(c * d_per, d_per)]
            ys = dy_ref[:, pl.ds(direction * f_half, f_half)]
            return lax.dot_general(
                xs, ys,
                dimension_numbers=(((0,), (0,)), ((), ())),
                preferred_element_type=jnp.float32,
            )

        def send_chunk(direction, s):
            k = pos - s - 1 if direction == CW else pos + s + 1
            return ring_at(k)

        def rdma(direction, s, j):
            tgt = right if direction == CW else left
            return pltpu.make_async_remote_copy(
                src_ref=comm_ref.at[direction, s, :, pl.ds(j * f_sub, f_sub)],
                dst_ref=comm_ref.at[
                    direction, s + 1, :, pl.ds(j * f_sub, f_sub)
                ],
                send_sem=send_sems.at[direction, s, j],
                recv_sem=recv_sems.at[direction, s, j],
                device_id=(tgt,),
                device_id_type=pl.DeviceIdType.MESH,
            )

        live = {}
        for direction in (CW, CCW):
            comm_ref[direction, 0, :, :] = contrib(send_chunk(direction, 0), direction)
            for j in range(SUB):
                r = rdma(direction, 0, j)
                r.start()
                live[(direction, 0, j)] = r

        nxt = {
            direction: contrib(send_chunk(direction, 1), direction)
            for direction in (CW, CCW)
        }

        for s in range(N_DEV - 1):
            last = s == N_DEV - 2
            cur = nxt
            for j in range(SUB):
                for direction in (CW, CCW):
                    live.pop((direction, s, j)).wait()
                    acc = (
                        comm_ref[direction, s + 1, :, pl.ds(j * f_sub, f_sub)]
                        + cur[direction][:, j * f_sub : (j + 1) * f_sub]
                    )
                    if last:
                        out_ref[
                            :, pl.ds(direction * f_half + j * f_sub, f_sub)
                        ] = acc
                    else:
                        comm_ref[
                            direction, s + 1, :, pl.ds(j * f_sub, f_sub)
                        ] = acc
                        r = rdma(direction, s + 1, j)
                        r.start()
                        live[(direction, s + 1, j)] = r
            if not last:
                nxt = {
                    direction: contrib(send_chunk(direction, s + 2), direction)
                    for direction in (CW, CCW)
                }

    return pl.pallas_call(
        body,
        out_shape=jax.ShapeDtypeStruct((d_per, f), jnp.float32),
        in_specs=[
            pl.BlockSpec(memory_space=pltpu.VMEM),
            pl.BlockSpec(memory_space=pltpu.VMEM),
        ],
        out_specs=pl.BlockSpec(memory_space=pltpu.VMEM),
        scratch_shapes=[
            pltpu.VMEM((2, N_DEV, d_per, f_half), jnp.float32),
            pltpu.SemaphoreType.DMA((2, N_DEV - 1, SUB)),
            pltpu.SemaphoreType.DMA((2, N_DEV - 1, SUB)),
        ],
        compiler_params=pltpu.CompilerParams(
            collective_id=0,
            vmem_limit_bytes=100 * 1024 * 1024,
        ),
    )(x, dy)


# device time: 96580 ns/iter; 1.0186x vs baseline; 1.0186x over previous
import jax
import jax.numpy as jnp
from jax import lax
from jax.experimental import pallas as pl
from jax.experimental.pallas import tpu as pltpu

N_DEV = 8
CW, CCW = 0, 1
SUB = 2
RING = (0, 1, 2, 3, 7, 6, 5, 4)


def kernel(x, dy):
    m, d = x.shape
    _, f = dy.shape
    d_per = d // N_DEV
    f_half = f // 2
    d_sub = d_per // SUB

    def body(x_ref, dy_ref, out_ref, comm_ref, send_sems, recv_sems):
        my = lax.axis_index("i")

        idx = lax.broadcasted_iota(jnp.int32, (1, N_DEV), 1)
        ring_arr = jnp.where(idx < 4, idx, 11 - idx)
        pos = jnp.sum(jnp.where(ring_arr == my, idx, 0))

        def ring_at(k):
            return jnp.sum(jnp.where(idx == (k % N_DEV), ring_arr, 0))

        right = ring_at(pos + 1)
        left = ring_at(pos - 1)

        def contrib(c, direction):
            xs = x_ref[:, pl.ds(c * d_per, d_per)]
            ys = dy_ref[:, pl.ds(direction * f_half, f_half)]
            return lax.dot_general(
                xs, ys,
                dimension_numbers=(((0,), (0,)), ((), ())),
                preferred_element_type=jnp.float32,
            )

        def send_chunk(direction, s):
            k = pos - s - 1 if direction == CW else pos + s + 1
            return ring_at(k)

        def rdma(direction, s, j):
            tgt = right if direction == CW else left
            return pltpu.make_async_remote_copy(
                src_ref=comm_ref.at[direction, s, pl.ds(j * d_sub, d_sub), :],
                dst_ref=comm_ref.at[
                    direction, s + 1, pl.ds(j * d_sub, d_sub), :
                ],
                send_sem=send_sems.at[direction, s, j],
                recv_sem=recv_sems.at[direction, s, j],
                device_id=(tgt,),
                device_id_type=pl.DeviceIdType.MESH,
            )

        barrier_sem = pltpu.get_barrier_semaphore()
        for nbr in (left, right):
            pl.semaphore_signal(
                barrier_sem, inc=1,
                device_id=(nbr,), device_id_type=pl.DeviceIdType.MESH,
            )
        for direction in (CW, CCW):
            comm_ref[direction, 0, :, :] = contrib(send_chunk(direction, 0), direction)
        pl.semaphore_wait(barrier_sem, 2)

        live = {}
        for direction in (CW, CCW):
            for j in range(SUB):
                r = rdma(direction, 0, j)
                r.start()
                live[(direction, 0, j)] = r

        nxt = {
            direction: contrib(send_chunk(direction, 1), direction)
            for direction in (CW, CCW)
        }

        for s in range(N_DEV - 1):
            last = s == N_DEV - 2
            cur = nxt
            for j in range(SUB):
                for direction in (CW, CCW):
                    live.pop((direction, s, j)).wait()
                    acc = (
                        comm_ref[direction, s + 1, pl.ds(j * d_sub, d_sub), :]
                        + cur[direction][j * d_sub : (j + 1) * d_sub, :]
                    )
                    if last:
                        out_ref[
                            pl.ds(j * d_sub, d_sub),
                            pl.ds(direction * f_half, f_half),
                        ] = acc
                    else:
                        comm_ref[
                            direction, s + 1, pl.ds(j * d_sub, d_sub), :
                        ] = acc
                        r = rdma(direction, s + 1, j)
                        r.start()
                        live[(direction, s + 1, j)] = r
            if not last:
                nxt = {
                    direction: contrib(send_chunk(direction, s + 2), direction)
                    for direction in (CW, CCW)
                }

    return pl.pallas_call(
        body,
        out_shape=jax.ShapeDtypeStruct((d_per, f), jnp.float32),
        in_specs=[
            pl.BlockSpec(memory_space=pltpu.VMEM),
            pl.BlockSpec(memory_space=pltpu.VMEM),
        ],
        out_specs=pl.BlockSpec(memory_space=pltpu.VMEM),
        scratch_shapes=[
            pltpu.VMEM((2, N_DEV, d_per, f_half), jnp.float32),
            pltpu.SemaphoreType.DMA((2, N_DEV - 1, SUB)),
            pltpu.SemaphoreType.DMA((2, N_DEV - 1, SUB)),
        ],
        compiler_params=pltpu.CompilerParams(
            collective_id=0,
            vmem_limit_bytes=100 * 1024 * 1024,
        ),
    )(x, dy)
